# Initial kernel scaffold; baseline (speedup 1.0000x reference)
import functools

import jax
import jax.numpy as jnp
from jax import lax
from jax.experimental import pallas as pl
from jax.experimental.pallas import tpu as pltpu
from jax.experimental.pallas import tpu_sc as plsc

_N = 10000
_E = 320000
_INDIM = 128
_OUTDIM = 32
_HEADS = 4

_NPAD = 10240
_XW = 144
_NC = 2
_NS = 16
_EPW = _E // (_NC * _NS)
_CH = 80
_NCHUNK = _EPW // _CH
_ROWS_PER_SUB = _NPAD // _NS


def _prep_body(xp_ref, np_ref, wc_ref, ac_ref, x_out_ref):
    hs = xp_ref[...] * np_ref[...]
    z = jnp.dot(hs, wc_ref[...], preferred_element_type=jnp.float32)
    sb = jnp.dot(z, ac_ref[...], preferred_element_type=jnp.float32)
    zext = jnp.concatenate(
        [z,
         jnp.ones((128, _HEADS), jnp.float32),
         jnp.zeros((128, _XW - _INDIM - _HEADS), jnp.float32)],
        axis=1)
    x_out_ref[...] = zext * jnp.exp(sb)


def _prep(xp, normp, wc, ac):
    grid = (_NPAD // 128,)
    return pl.pallas_call(
        _prep_body,
        grid=grid,
        in_specs=[
            pl.BlockSpec((128, _INDIM), lambda i: (i, 0)),
            pl.BlockSpec((128, 1), lambda i: (i, 0)),
            pl.BlockSpec((_INDIM, _INDIM), lambda i: (0, 0)),
            pl.BlockSpec((_INDIM, _XW), lambda i: (0, 0)),
        ],
        out_specs=pl.BlockSpec((128, _XW), lambda i: (i, 0)),
        out_shape=jax.ShapeDtypeStruct((_NPAD, _XW), jnp.float32),
    )(xp, normp, wc, ac)


def _sc_scatter(src_idx, dst_idx, xtab, zrows):
    mesh = plsc.VectorSubcoreMesh(core_axis_name="c", subcore_axis_name="s")

    @functools.partial(
        pl.kernel,
        out_type=jax.ShapeDtypeStruct((_NC, _NPAD, _XW), jnp.float32),
        mesh=mesh,
        scratch_types=[
            pltpu.VMEM((_NCHUNK, _CH), jnp.int32),
            pltpu.VMEM((_NCHUNK, _CH), jnp.int32),
            pltpu.VMEM((_CH, _XW), jnp.float32),
            pltpu.VMEM((128, _XW), jnp.float32),
            pltpu.VMEM_SHARED((_NPAD, _XW), jnp.float32),
        ],
    )
    def sc_kernel(src_hbm, dst_hbm, x_hbm, z_hbm, out_hbm,
                  srcv, dstv, rows, zbuf, acc):
        c = lax.axis_index("c")
        s = lax.axis_index("s")

        pltpu.sync_copy(z_hbm, zbuf)

        @pl.loop(0, _ROWS_PER_SUB // 128)
        def _(k):
            pltpu.sync_copy(zbuf, acc.at[pl.ds(s * _ROWS_PER_SUB + k * 128, 128)])

        plsc.subcore_barrier()

        pltpu.sync_copy(src_hbm.at[c].at[s], srcv)
        pltpu.sync_copy(dst_hbm.at[c].at[s], dstv)

        @pl.loop(0, _NCHUNK)
        def _(j):
            pltpu.sync_copy(x_hbm.at[srcv.at[j]], rows)
            pltpu.sync_copy(rows, acc.at[dstv.at[j]], add=True)

        plsc.subcore_barrier()

        pltpu.sync_copy(acc.at[pl.ds(s * _ROWS_PER_SUB, _ROWS_PER_SUB)],
                        out_hbm.at[c].at[pl.ds(s * _ROWS_PER_SUB, _ROWS_PER_SUB)])

    return sc_kernel(src_idx, dst_idx, xtab, zrows)


def _fin_body(yn_ref, yd_ref, xp_ref, np_ref, ps_ref, out_ref):
    y = yn_ref[0] + yn_ref[1]
    dh = yd_ref[0] + yd_ref[1]
    d32 = jnp.dot(dh, ps_ref[...], preferred_element_type=jnp.float32)
    out_ref[...] = xp_ref[...] + np_ref[...] * y / jnp.maximum(d32, 1e-16)


def _finalize(ypart, xp, normp, psel):
    grid = (_NPAD // 128,)
    return pl.pallas_call(
        _fin_body,
        grid=grid,
        in_specs=[
            pl.BlockSpec((_NC, 128, _INDIM), lambda i: (0, i, 0)),
            pl.BlockSpec((_NC, 128, 16), lambda i: (0, i, _INDIM // 16)),
            pl.BlockSpec((128, _INDIM), lambda i: (i, 0)),
            pl.BlockSpec((128, 1), lambda i: (i, 0)),
            pl.BlockSpec((16, _INDIM), lambda i: (0, 0)),
        ],
        out_specs=pl.BlockSpec((128, _INDIM), lambda i: (i, 0)),
        out_shape=jax.ShapeDtypeStruct((_NPAD, _INDIM), jnp.float32),
    )(ypart, ypart, xp, normp, psel)


def kernel(x, edge_index, e, norm, W, a):
    f32 = jnp.float32

    wc = jnp.transpose(W, (1, 0, 2)).reshape(_INDIM, _HEADS * _OUTDIM)
    a_src_flat = a[:, :_OUTDIM, 0].reshape(_HEADS * _OUTDIM)
    rows_i = jnp.arange(_INDIM)[:, None]
    cols_i = jnp.arange(_XW)[None, :]
    mask_main = (cols_i < _INDIM) & (rows_i // _OUTDIM == cols_i // _OUTDIM)
    mask_den = ((cols_i >= _INDIM) & (cols_i < _INDIM + _HEADS)
                & (rows_i // _OUTDIM == cols_i - _INDIM))
    ac = jnp.where(mask_main | mask_den, a_src_flat[:, None], 0.0).astype(f32)

    prow = jnp.arange(16)[:, None]
    pcol = jnp.arange(_INDIM)[None, :]
    psel = ((pcol // _OUTDIM == prow) & (prow < _HEADS)).astype(f32)

    xp = jnp.pad(x, ((0, _NPAD - _N), (0, 0)))
    normp = jnp.pad(norm, ((0, _NPAD - _N), (0, 0)))
    src_idx = edge_index[0].reshape(_NC, _NS, _NCHUNK, _CH)
    dst_idx = edge_index[1].reshape(_NC, _NS, _NCHUNK, _CH)
    zrows = jnp.zeros((128, _XW), f32)

    xtab = _prep(xp, normp, wc, ac)
    ypart = _sc_scatter(src_idx, dst_idx, xtab, zrows)
    out = _finalize(ypart, xp, normp, psel)
    return (out[:_N], e)


# speedup vs baseline: 59.0937x; 59.0937x over previous
import functools

import jax
import jax.numpy as jnp
from jax import lax
from jax.experimental import pallas as pl
from jax.experimental.pallas import tpu as pltpu
from jax.experimental.pallas import tpu_sc as plsc

_N = 10000
_E = 320000
_INDIM = 128
_OUTDIM = 32
_HEADS = 4

_NPAD = 10240
_XW = 144
_NC = 2
_NS = 16
_EPW = _E // (_NC * _NS)
_CH = 80
_NCHUNK = _EPW // _CH
_ROWS_PER_SUB = _NPAD // _NS


def _prep_body(xp_ref, np_ref, wc_ref, ac_ref, x_out_ref):
    hs = xp_ref[...] * np_ref[...]
    z = jnp.dot(hs, wc_ref[...], preferred_element_type=jnp.float32)
    sb = jnp.dot(z, ac_ref[...], preferred_element_type=jnp.float32)
    zext = jnp.concatenate(
        [z,
         jnp.ones((128, _HEADS), jnp.float32),
         jnp.zeros((128, _XW - _INDIM - _HEADS), jnp.float32)],
        axis=1)
    x_out_ref[...] = zext * jnp.exp(sb)


def _prep(xp, normp, wc, ac):
    grid = (_NPAD // 128,)
    return pl.pallas_call(
        _prep_body,
        grid=grid,
        in_specs=[
            pl.BlockSpec((128, _INDIM), lambda i: (i, 0)),
            pl.BlockSpec((128, 1), lambda i: (i, 0)),
            pl.BlockSpec((_INDIM, _INDIM), lambda i: (0, 0)),
            pl.BlockSpec((_INDIM, _XW), lambda i: (0, 0)),
        ],
        out_specs=pl.BlockSpec((128, _XW), lambda i: (i, 0)),
        out_shape=jax.ShapeDtypeStruct((_NPAD, _XW), jnp.float32),
    )(xp, normp, wc, ac)


def _sc_scatter(src_idx, dst_idx, xtab, zrows):
    mesh = plsc.VectorSubcoreMesh(core_axis_name="c", subcore_axis_name="s")

    @functools.partial(
        pl.kernel,
        out_type=jax.ShapeDtypeStruct((_NC, _NPAD, _XW), jnp.float32),
        mesh=mesh,
        scratch_types=[
            pltpu.VMEM((_NCHUNK, _CH), jnp.int32),
            pltpu.VMEM((_NCHUNK, _CH), jnp.int32),
            pltpu.VMEM((_CH, _XW), jnp.float32),
            pltpu.VMEM_SHARED((_NPAD, _XW), jnp.float32),
        ],
        compiler_params=pltpu.CompilerParams(use_tc_tiling_on_sc=False),
    )
    def sc_kernel(src_hbm, dst_hbm, x_hbm, z_hbm, out_hbm,
                  srcv, dstv, rows, acc):
        c = lax.axis_index("c")
        s = lax.axis_index("s")

        pltpu.sync_copy(z_hbm, acc.at[pl.ds(s * _ROWS_PER_SUB, _ROWS_PER_SUB)])

        plsc.subcore_barrier()

        pltpu.sync_copy(src_hbm.at[c].at[s], srcv)
        pltpu.sync_copy(dst_hbm.at[c].at[s], dstv)

        @pl.loop(0, _NCHUNK)
        def _(j):
            pltpu.sync_copy(x_hbm.at[srcv.at[j]], rows)
            pltpu.sync_copy(rows, acc.at[dstv.at[j]], add=True)

        plsc.subcore_barrier()

        pltpu.sync_copy(acc.at[pl.ds(s * _ROWS_PER_SUB, _ROWS_PER_SUB)],
                        out_hbm.at[c].at[pl.ds(s * _ROWS_PER_SUB, _ROWS_PER_SUB)])

    return sc_kernel(src_idx, dst_idx, xtab, zrows)


def _fin_body(y_ref, xp_ref, np_ref, ps_ref, out_ref):
    yfull = y_ref[0] + y_ref[1]
    y = yfull[:, :_INDIM]
    dh = yfull[:, _INDIM:]
    d32 = jnp.dot(dh, ps_ref[...], preferred_element_type=jnp.float32)
    out_ref[...] = xp_ref[...] + np_ref[...] * y / jnp.maximum(d32, 1e-16)


def _finalize(ypart, xp, normp, psel):
    grid = (_NPAD // 128,)
    return pl.pallas_call(
        _fin_body,
        grid=grid,
        in_specs=[
            pl.BlockSpec((_NC, 128, _XW), lambda i: (0, i, 0)),
            pl.BlockSpec((128, _INDIM), lambda i: (i, 0)),
            pl.BlockSpec((128, 1), lambda i: (i, 0)),
            pl.BlockSpec((16, _INDIM), lambda i: (0, 0)),
        ],
        out_specs=pl.BlockSpec((128, _INDIM), lambda i: (i, 0)),
        out_shape=jax.ShapeDtypeStruct((_NPAD, _INDIM), jnp.float32),
    )(ypart, xp, normp, psel)


def kernel(x, edge_index, e, norm, W, a):
    f32 = jnp.float32

    wc = jnp.transpose(W, (1, 0, 2)).reshape(_INDIM, _HEADS * _OUTDIM)
    a_src_flat = a[:, :_OUTDIM, 0].reshape(_HEADS * _OUTDIM)
    rows_i = jnp.arange(_INDIM)[:, None]
    cols_i = jnp.arange(_XW)[None, :]
    mask_main = (cols_i < _INDIM) & (rows_i // _OUTDIM == cols_i // _OUTDIM)
    mask_den = ((cols_i >= _INDIM) & (cols_i < _INDIM + _HEADS)
                & (rows_i // _OUTDIM == cols_i - _INDIM))
    ac = jnp.where(mask_main | mask_den, a_src_flat[:, None], 0.0).astype(f32)

    prow = jnp.arange(16)[:, None]
    pcol = jnp.arange(_INDIM)[None, :]
    psel = ((pcol // _OUTDIM == prow) & (prow < _HEADS)).astype(f32)

    xp = jnp.pad(x, ((0, _NPAD - _N), (0, 0)))
    normp = jnp.pad(norm, ((0, _NPAD - _N), (0, 0)))
    src_idx = edge_index[0].reshape(_NC, _NS, _NCHUNK, _CH)
    dst_idx = edge_index[1].reshape(_NC, _NS, _NCHUNK, _CH)
    zrows = jnp.zeros((_ROWS_PER_SUB, _XW), f32)

    xtab = _prep(xp, normp, wc, ac)
    ypart = _sc_scatter(src_idx, dst_idx, xtab, zrows)
    out = _finalize(ypart, xp, normp, psel)
    return (out[:_N], e)


# speedup vs baseline: 108.1772x; 1.8306x over previous
import functools

import jax
import jax.numpy as jnp
from jax import lax
from jax.experimental import pallas as pl
from jax.experimental.pallas import tpu as pltpu
from jax.experimental.pallas import tpu_sc as plsc

_N = 10000
_E = 320000
_INDIM = 128
_OUTDIM = 32
_HEADS = 4

_NPAD = 10240
_XW = 144
_NC = 2
_NS = 16
_EPW = _E // (_NC * _NS)
_CH = 80
_NCHUNK = _EPW // _CH
_MP = 62
_ROWS_PER_SUB = _NPAD // _NS
_RB = 2000


def _prep_body(x_ref, n_ref, wc_ref, ac_ref, x_out_ref):
    hs = x_ref[...] * n_ref[...]
    z = jnp.dot(hs, wc_ref[...], preferred_element_type=jnp.float32)
    sb = jnp.dot(z, ac_ref[...], preferred_element_type=jnp.float32)
    zext = jnp.concatenate(
        [z,
         jnp.ones((_RB, _HEADS), jnp.float32),
         jnp.zeros((_RB, _XW - _INDIM - _HEADS), jnp.float32)],
        axis=1)
    x_out_ref[...] = zext * jnp.exp(sb)


def _prep(x, norm, wc, ac):
    grid = (_N // _RB,)
    return pl.pallas_call(
        _prep_body,
        grid=grid,
        in_specs=[
            pl.BlockSpec((_RB, _INDIM), lambda i: (i, 0)),
            pl.BlockSpec((_RB, 1), lambda i: (i, 0)),
            pl.BlockSpec((_INDIM, _INDIM), lambda i: (0, 0)),
            pl.BlockSpec((_INDIM, _XW), lambda i: (0, 0)),
        ],
        out_specs=pl.BlockSpec((_RB, _XW), lambda i: (i, 0)),
        out_shape=jax.ShapeDtypeStruct((_N, _XW), jnp.float32),
    )(x, norm, wc, ac)


def _sc_scatter(edge_r, xtab, zrows):
    mesh = plsc.VectorSubcoreMesh(core_axis_name="c", subcore_axis_name="s")

    @functools.partial(
        pl.kernel,
        out_type=jax.ShapeDtypeStruct((_NC, _NPAD, _XW), jnp.float32),
        mesh=mesh,
        scratch_types=[
            pltpu.VMEM((_MP, _CH), jnp.int32),
            pltpu.VMEM((_MP, _CH), jnp.int32),
            pltpu.VMEM((_CH, _XW), jnp.float32),
            pltpu.VMEM((_CH, _XW), jnp.float32),
            pltpu.VMEM_SHARED((_NPAD, _XW), jnp.float32),
            pltpu.SemaphoreType.DMA,
            pltpu.SemaphoreType.DMA,
        ],
        compiler_params=pltpu.CompilerParams(use_tc_tiling_on_sc=False),
    )
    def sc_kernel(edge_hbm, x_hbm, z_hbm, out_hbm,
                  srcv, dstv, rows0, rows1, acc, sem0, sem1):
        c = lax.axis_index("c")
        s = lax.axis_index("s")

        pltpu.sync_copy(z_hbm, acc.at[pl.ds(s * _ROWS_PER_SUB, _ROWS_PER_SUB)])

        plsc.subcore_barrier()

        def do_pass(base):
            pltpu.sync_copy(edge_hbm.at[0].at[c].at[s].at[pl.ds(base, _MP)], srcv)
            pltpu.sync_copy(edge_hbm.at[1].at[c].at[s].at[pl.ds(base, _MP)], dstv)

            pltpu.async_copy(x_hbm.at[srcv.at[0]], rows0, sem0)

            @pl.loop(0, _MP - 2, step=2)
            def _(j):
                pltpu.async_copy(x_hbm.at[srcv.at[j + 1]], rows1, sem1)
                pltpu.make_async_copy(x_hbm.at[srcv.at[j]], rows0, sem0).wait()
                pltpu.sync_copy(rows0, acc.at[dstv.at[j]], add=True)
                pltpu.async_copy(x_hbm.at[srcv.at[j + 2]], rows0, sem0)
                pltpu.make_async_copy(x_hbm.at[srcv.at[j + 1]], rows1, sem1).wait()
                pltpu.sync_copy(rows1, acc.at[dstv.at[j + 1]], add=True)

            pltpu.async_copy(x_hbm.at[srcv.at[_MP - 1]], rows1, sem1)
            pltpu.make_async_copy(x_hbm.at[srcv.at[_MP - 2]], rows0, sem0).wait()
            pltpu.sync_copy(rows0, acc.at[dstv.at[_MP - 2]], add=True)
            pltpu.make_async_copy(x_hbm.at[srcv.at[_MP - 1]], rows1, sem1).wait()
            pltpu.sync_copy(rows1, acc.at[dstv.at[_MP - 1]], add=True)

        do_pass(0)
        do_pass(_MP)

        pltpu.sync_copy(edge_hbm.at[0].at[c].at[s].at[pl.ds(2 * _MP, 1)],
                        srcv.at[pl.ds(0, 1)])
        pltpu.sync_copy(edge_hbm.at[1].at[c].at[s].at[pl.ds(2 * _MP, 1)],
                        dstv.at[pl.ds(0, 1)])
        pltpu.sync_copy(x_hbm.at[srcv.at[0]], rows0)
        pltpu.sync_copy(rows0, acc.at[dstv.at[0]], add=True)

        plsc.subcore_barrier()

        pltpu.sync_copy(acc.at[pl.ds(s * _ROWS_PER_SUB, _ROWS_PER_SUB)],
                        out_hbm.at[c].at[pl.ds(s * _ROWS_PER_SUB, _ROWS_PER_SUB)])

    return sc_kernel(edge_r, xtab, zrows)


def _fin_body(y_ref, x_ref, n_ref, ps_ref, out_ref):
    yfull = y_ref[0] + y_ref[1]
    y = yfull[:, :_INDIM]
    dh = yfull[:, _INDIM:]
    d32 = jnp.dot(dh, ps_ref[...], preferred_element_type=jnp.float32)
    out_ref[...] = x_ref[...] + n_ref[...] * y / jnp.maximum(d32, 1e-16)


def _finalize(ypart, x, norm, psel):
    grid = (_N // _RB,)
    return pl.pallas_call(
        _fin_body,
        grid=grid,
        in_specs=[
            pl.BlockSpec((_NC, _RB, _XW), lambda i: (0, i, 0)),
            pl.BlockSpec((_RB, _INDIM), lambda i: (i, 0)),
            pl.BlockSpec((_RB, 1), lambda i: (i, 0)),
            pl.BlockSpec((16, _INDIM), lambda i: (0, 0)),
        ],
        out_specs=pl.BlockSpec((_RB, _INDIM), lambda i: (i, 0)),
        out_shape=jax.ShapeDtypeStruct((_N, _INDIM), jnp.float32),
    )(ypart, x, norm, psel)


def kernel(x, edge_index, e, norm, W, a):
    f32 = jnp.float32

    wc = jnp.transpose(W, (1, 0, 2)).reshape(_INDIM, _HEADS * _OUTDIM)
    a_src_flat = a[:, :_OUTDIM, 0].reshape(_HEADS * _OUTDIM)
    rows_i = jnp.arange(_INDIM)[:, None]
    cols_i = jnp.arange(_XW)[None, :]
    mask_main = (cols_i < _INDIM) & (rows_i // _OUTDIM == cols_i // _OUTDIM)
    mask_den = ((cols_i >= _INDIM) & (cols_i < _INDIM + _HEADS)
                & (rows_i // _OUTDIM == cols_i - _INDIM))
    ac = jnp.where(mask_main | mask_den, a_src_flat[:, None], 0.0).astype(f32)

    prow = jnp.arange(16)[:, None]
    pcol = jnp.arange(_INDIM)[None, :]
    psel = ((pcol // _OUTDIM == prow) & (prow < _HEADS)).astype(f32)

    edge_r = edge_index.reshape(2, _NC, _NS, _NCHUNK, _CH)
    zrows = jnp.zeros((_ROWS_PER_SUB, _XW), f32)

    xtab = _prep(x, norm, wc, ac)
    ypart = _sc_scatter(edge_r, xtab, zrows)
    out = _finalize(ypart, x, norm, psel)
    return (out, e)


# speedup vs baseline: 110.0104x; 1.0169x over previous
import functools

import jax
import jax.numpy as jnp
from jax import lax
from jax.experimental import pallas as pl
from jax.experimental.pallas import tpu as pltpu
from jax.experimental.pallas import tpu_sc as plsc

_N = 10000
_E = 320000
_INDIM = 128
_OUTDIM = 32
_HEADS = 4

_NPAD = 10000
_XW = 144
_NC = 2
_NS = 16
_EPW = _E // (_NC * _NS)
_CH = 80
_NCHUNK = _EPW // _CH
_MP = 62
_LEFT = _NCHUNK - 2 * _MP
_ROWS_PER_SUB = _NPAD // _NS
_RB = 2000


def _prep_body(x_ref, n_ref, wc_ref, ac_ref, x_out_ref):
    hs = x_ref[...] * n_ref[...]
    z = jnp.dot(hs, wc_ref[...], preferred_element_type=jnp.float32)
    sb = jnp.dot(z, ac_ref[...], preferred_element_type=jnp.float32)
    zext = jnp.concatenate(
        [z,
         jnp.ones((_RB, _HEADS), jnp.float32),
         jnp.zeros((_RB, _XW - _INDIM - _HEADS), jnp.float32)],
        axis=1)
    x_out_ref[...] = zext * jnp.exp(sb)


def _prep(x, norm, wc, ac):
    grid = (_N // _RB,)
    return pl.pallas_call(
        _prep_body,
        grid=grid,
        in_specs=[
            pl.BlockSpec((_RB, _INDIM), lambda i: (i, 0)),
            pl.BlockSpec((_RB, 1), lambda i: (i, 0)),
            pl.BlockSpec((_INDIM, _INDIM), lambda i: (0, 0)),
            pl.BlockSpec((_INDIM, _XW), lambda i: (0, 0)),
        ],
        out_specs=pl.BlockSpec((_RB, _XW), lambda i: (i, 0)),
        out_shape=jax.ShapeDtypeStruct((_N, _XW), jnp.float32),
    )(x, norm, wc, ac)


def _sc_scatter(edge_r, xtab):
    mesh = plsc.VectorSubcoreMesh(core_axis_name="c", subcore_axis_name="s")

    @functools.partial(
        pl.kernel,
        out_type=jax.ShapeDtypeStruct((_NC, _NPAD, _XW), jnp.float32),
        mesh=mesh,
        scratch_types=[
            pltpu.VMEM((_MP, _CH), jnp.int32),
            pltpu.VMEM((_MP, _CH), jnp.int32),
            pltpu.VMEM((_CH, _XW), jnp.float32),
            pltpu.VMEM((_CH, _XW), jnp.float32),
            pltpu.VMEM_SHARED((_NPAD, _XW), jnp.float32),
            pltpu.SemaphoreType.DMA,
            pltpu.SemaphoreType.DMA,
            pltpu.SemaphoreType.DMA,
            pltpu.SemaphoreType.DMA,
        ],
        compiler_params=pltpu.CompilerParams(use_tc_tiling_on_sc=False),
    )
    def sc_kernel(edge_hbm, x_hbm, out_hbm,
                  srcv, dstv, b0, b1, acc, g0, g1, s0, s1):
        c = lax.axis_index("c")
        s = lax.axis_index("s")
        bufs = (b0, b1)
        gsem = (g0, g1)
        ssem = (s0, s1)
        my0 = s * _ROWS_PER_SUB

        def g_start(j, k):
            pltpu.async_copy(x_hbm.at[srcv.at[j]], bufs[k], gsem[k])

        def g_wait(j, k):
            pltpu.make_async_copy(x_hbm.at[srcv.at[j]], bufs[k], gsem[k]).wait()

        def s_start(j, k):
            pltpu.async_copy(bufs[k], acc.at[dstv.at[j]], ssem[k], add=True)

        def s_wait(j, k):
            pltpu.make_async_copy(bufs[k], acc.at[dstv.at[j]], ssem[k]).wait()

        @pl.loop(0, _CH)
        def _(i):
            @pl.loop(0, _XW // 16)
            def _(kk):
                b0.at[i].at[pl.ds(kk * 16, 16)][...] = jnp.zeros((16,), jnp.float32)

        for r in range(7):
            pltpu.sync_copy(b0, acc.at[pl.ds(my0 + r * _CH, _CH)])
        pltpu.sync_copy(b0.at[pl.ds(0, 65)], acc.at[pl.ds(my0 + 560, 65)])

        plsc.subcore_barrier()

        @pl.loop(0, 2)
        def _(p):
            base = p * _MP
            pltpu.sync_copy(edge_hbm.at[0].at[c].at[s].at[pl.ds(base, _MP)], srcv)
            pltpu.sync_copy(edge_hbm.at[1].at[c].at[s].at[pl.ds(base, _MP)], dstv)

            g_start(0, 0)

            @pl.loop(0, _MP, step=2)
            def _(j):
                for o in range(2):
                    ch = j + o
                    ko = (o + 1) % 2

                    @pl.when(ch >= 1)
                    def _():
                        s_wait(ch - 1, ko)

                    @pl.when(ch + 1 < _MP)
                    def _():
                        g_start(ch + 1, ko)

                    g_wait(ch, o)
                    s_start(ch, o)

            s_wait(_MP - 1, (_MP - 1) % 2)

        pltpu.sync_copy(edge_hbm.at[0].at[c].at[s].at[pl.ds(2 * _MP, _LEFT)],
                        srcv.at[pl.ds(0, _LEFT)])
        pltpu.sync_copy(edge_hbm.at[1].at[c].at[s].at[pl.ds(2 * _MP, _LEFT)],
                        dstv.at[pl.ds(0, _LEFT)])

        @pl.loop(0, _LEFT)
        def _(jj):
            pltpu.sync_copy(x_hbm.at[srcv.at[jj]], b0)
            pltpu.sync_copy(b0, acc.at[dstv.at[jj]], add=True)

        plsc.subcore_barrier()

        pltpu.sync_copy(acc.at[pl.ds(my0, _ROWS_PER_SUB)],
                        out_hbm.at[c].at[pl.ds(my0, _ROWS_PER_SUB)])

    return sc_kernel(edge_r, xtab)


def _fin_body(y_ref, x_ref, n_ref, ps_ref, out_ref):
    yfull = y_ref[0] + y_ref[1]
    y = yfull[:, :_INDIM]
    dh = yfull[:, _INDIM:]
    d32 = jnp.dot(dh, ps_ref[...], preferred_element_type=jnp.float32)
    out_ref[...] = x_ref[...] + n_ref[...] * y / jnp.maximum(d32, 1e-16)


def _finalize(ypart, x, norm, psel):
    grid = (_N // _RB,)
    return pl.pallas_call(
        _fin_body,
        grid=grid,
        in_specs=[
            pl.BlockSpec((_NC, _RB, _XW), lambda i: (0, i, 0)),
            pl.BlockSpec((_RB, _INDIM), lambda i: (i, 0)),
            pl.BlockSpec((_RB, 1), lambda i: (i, 0)),
            pl.BlockSpec((16, _INDIM), lambda i: (0, 0)),
        ],
        out_specs=pl.BlockSpec((_RB, _INDIM), lambda i: (i, 0)),
        out_shape=jax.ShapeDtypeStruct((_N, _INDIM), jnp.float32),
    )(ypart, x, norm, psel)


def kernel(x, edge_index, e, norm, W, a):
    f32 = jnp.float32

    wc = jnp.transpose(W, (1, 0, 2)).reshape(_INDIM, _HEADS * _OUTDIM)
    a_src_flat = a[:, :_OUTDIM, 0].reshape(_HEADS * _OUTDIM)
    rows_i = jnp.arange(_INDIM)[:, None]
    cols_i = jnp.arange(_XW)[None, :]
    mask_main = (cols_i < _INDIM) & (rows_i // _OUTDIM == cols_i // _OUTDIM)
    mask_den = ((cols_i >= _INDIM) & (cols_i < _INDIM + _HEADS)
                & (rows_i // _OUTDIM == cols_i - _INDIM))
    ac = jnp.where(mask_main | mask_den, a_src_flat[:, None], 0.0).astype(f32)

    prow = jnp.arange(16)[:, None]
    pcol = jnp.arange(_INDIM)[None, :]
    psel = ((pcol // _OUTDIM == prow) & (prow < _HEADS)).astype(f32)

    edge_r = edge_index.reshape(2, _NC, _NS, _NCHUNK, _CH)

    xtab = _prep(x, norm, wc, ac)
    ypart = _sc_scatter(edge_r, xtab)
    out = _finalize(ypart, x, norm, psel)
    return (out, e)
